# Initial kernel scaffold; baseline (speedup 1.0000x reference)
import jax
import jax.numpy as jnp
from jax.experimental import pallas as pl


def kernel(x, W1, b1, W2, b2):
    raise NotImplementedError("write your pallas kernel here")


# speedup vs baseline: 1.5850x; 1.5850x over previous
import jax
import jax.numpy as jnp
from jax.experimental import pallas as pl
from jax.experimental.pallas import tpu as pltpu

_FEAT = 64
_EMB = 128
_BLOCK = 8000


def _fused_kernel(x_ref, w1_ref, b1_ref, w2_ref, b2_ref, o_ref):
    x = x_ref[...]
    h = jnp.dot(x, w1_ref[...], preferred_element_type=jnp.float32) + b1_ref[...]
    h = jnp.maximum(h, 0.0)
    e = jnp.dot(h, w2_ref[...], preferred_element_type=jnp.float32) + b2_ref[...]
    sq = jnp.sum(e * e, axis=-1, keepdims=True)
    inv = jax.lax.rsqrt(sq)
    o_ref[...] = jnp.where(sq > 0.0, e * inv, 0.0)


def kernel(x, W1, b1, W2, b2):
    n_rows = x.shape[0]
    grid = (n_rows // _BLOCK,)
    return pl.pallas_call(
        _fused_kernel,
        grid=grid,
        in_specs=[
            pl.BlockSpec((_BLOCK, _FEAT), lambda i: (i, 0)),
            pl.BlockSpec((_FEAT, _FEAT), lambda i: (0, 0)),
            pl.BlockSpec((1, _FEAT), lambda i: (0, 0)),
            pl.BlockSpec((_FEAT, _EMB), lambda i: (0, 0)),
            pl.BlockSpec((1, _EMB), lambda i: (0, 0)),
        ],
        out_specs=pl.BlockSpec((_BLOCK, _EMB), lambda i: (i, 0)),
        out_shape=jax.ShapeDtypeStruct((n_rows, _EMB), jnp.float32),
        compiler_params=pltpu.CompilerParams(
            dimension_semantics=("parallel",),
        ),
    )(x, W1, b1.reshape(1, _FEAT), W2, b2.reshape(1, _EMB))


# speedup vs baseline: 1.6167x; 1.0200x over previous
import jax
import jax.numpy as jnp
from jax.experimental import pallas as pl
from jax.experimental.pallas import tpu as pltpu

_FEAT = 64
_EMB = 128
_BLOCK = 20000


def _fused_kernel(x_ref, w1_ref, b1_ref, w2_ref, b2_ref, o_ref):
    x = x_ref[...]
    h = jnp.dot(x, w1_ref[...], preferred_element_type=jnp.float32) + b1_ref[...]
    h = jnp.maximum(h, 0.0)
    e = jnp.dot(h, w2_ref[...], preferred_element_type=jnp.float32) + b2_ref[...]
    sq = jnp.sum(e * e, axis=-1, keepdims=True)
    inv = jax.lax.rsqrt(sq)
    o_ref[...] = jnp.where(sq > 0.0, e * inv, 0.0)


def kernel(x, W1, b1, W2, b2):
    n_rows = x.shape[0]
    grid = (n_rows // _BLOCK,)
    return pl.pallas_call(
        _fused_kernel,
        grid=grid,
        in_specs=[
            pl.BlockSpec((_BLOCK, _FEAT), lambda i: (i, 0)),
            pl.BlockSpec((_FEAT, _FEAT), lambda i: (0, 0)),
            pl.BlockSpec((1, _FEAT), lambda i: (0, 0)),
            pl.BlockSpec((_FEAT, _EMB), lambda i: (0, 0)),
            pl.BlockSpec((1, _EMB), lambda i: (0, 0)),
        ],
        out_specs=pl.BlockSpec((_BLOCK, _EMB), lambda i: (i, 0)),
        out_shape=jax.ShapeDtypeStruct((n_rows, _EMB), jnp.float32),
        compiler_params=pltpu.CompilerParams(
            dimension_semantics=("arbitrary",),
            vmem_limit_bytes=56 * 1024 * 1024,
        ),
    )(x, W1, b1.reshape(1, _FEAT), W2, b2.reshape(1, _EMB))
